# baseline (speedup 1.0000x reference)
import functools
import jax
import jax.numpy as jnp
from jax import lax
from jax.experimental import pallas as pl
from jax.experimental.pallas import tpu as pltpu

B, M, D = 256, 2048, 128
BB = 8
NB = B // BB
NCH = 4
CH = M // NCH
BIG = 2 ** 30


def _attn_body(f0_ref, f1_ref, f2_ref, f3_ref, cls_ref, freq_ref,
               wv_ref, bv_ref, wu_ref, bu_ref, wa_ref, ba_ref,
               z_ref, feat_out_hbm, freq_out_ref, idx_ref,
               cls_stage, sems, tail_sem):
    b = pl.program_id(0)
    rows = pl.ds(b * BB, BB)
    chunks = [f0_ref, f1_ref, f2_ref, f3_ref]

    copies = []
    for k, ck in enumerate(chunks):
        dma = pltpu.make_async_copy(
            ck, feat_out_hbm.at[rows, pl.ds(k * CH, CH)], sems.at[k])
        dma.start()
        copies.append(dma)
    c = cls_ref[...]
    cls_stage[...] = c[:, None, :]
    tail = pltpu.make_async_copy(
        cls_stage, feat_out_hbm.at[rows, pl.ds(M, 1)], tail_sem)
    tail.start()

    wv = wv_ref[...]
    wu = wu_ref[...]
    wa = wa_ref[...]
    bv = bv_ref[...]
    bu = bu_ref[...]
    ba = ba_ref[0, 0]

    z_acc = None
    scores = []
    for k, ck in enumerate(chunks):
        fk = ck[...]
        fk2 = fk.reshape(BB * CH, D)
        v = jnp.tanh(jnp.dot(fk2, wv, preferred_element_type=jnp.float32) + bv)
        u = jax.nn.sigmoid(jnp.dot(fk2, wu,
                                   preferred_element_type=jnp.float32) + bu)
        logit = jnp.dot(v * u, wa, preferred_element_type=jnp.float32) + ba
        sk = jax.nn.sigmoid(logit).reshape(BB, CH)
        scores.append(sk)
        zc = lax.dot_general(sk.reshape(BB, 1, CH), fk,
                             (((2,), (1,)), ((0,), (0,))),
                             preferred_element_type=jnp.float32)
        zc = zc.reshape(BB, D)
        z_acc = zc if z_acc is None else z_acc + zc

    vc = jnp.tanh(jnp.dot(c, wv, preferred_element_type=jnp.float32) + bv)
    uc = jax.nn.sigmoid(jnp.dot(c, wu, preferred_element_type=jnp.float32) + bu)
    sc = jax.nn.sigmoid(jnp.dot(vc * uc, wa,
                                preferred_element_type=jnp.float32) + ba)

    z_ref[...] = z_acc + sc * c

    score = jnp.concatenate(scores, axis=1)
    gmin = jnp.min(score, axis=1, keepdims=True)
    iot = lax.broadcasted_iota(jnp.int32, (BB, M), 1)
    arg = jnp.min(jnp.where(score == gmin, iot, BIG), axis=1, keepdims=True)
    idx_ref[...] = jnp.where(sc < gmin, jnp.int32(M), arg)

    freq_out_ref[:, :M] = freq_ref[...] + 1
    freq_out_ref[:, M:] = jnp.ones((BB, 1), jnp.int32)

    for dma in copies:
        dma.wait()
    tail.wait()


def _min_body(min_ref, idx_ref, out_ref):
    idx = idx_ref[...]
    iot = lax.broadcasted_iota(jnp.int32, (BB, M), 1)
    out_ref[:, :M] = min_ref[...] + (iot == idx).astype(jnp.int32)
    out_ref[:, M:] = (idx == M).astype(jnp.int32)


def kernel(x, feat_mem, freq_mem, min_mem, is_last, W_v, b_v, W_u, b_u, W_a, b_a):
    cls = x[:, 0, :]
    wvT = W_v.T
    wuT = W_u.T
    wa = W_a.reshape(D, 1)
    ba = b_a.reshape(1, 1)
    bv = b_v.reshape(1, D)
    bu = b_u.reshape(1, D)

    feat_chunk_specs = [
        pl.BlockSpec((BB, CH, D), functools.partial(lambda k, b: (b, k, 0), k))
        for k in range(NCH)
    ]
    z, feat, freq_new, idx = pl.pallas_call(
        _attn_body,
        grid=(NB,),
        in_specs=feat_chunk_specs + [
            pl.BlockSpec((BB, D), lambda b: (b, 0)),
            pl.BlockSpec((BB, M), lambda b: (b, 0)),
            pl.BlockSpec((D, D), lambda b: (0, 0)),
            pl.BlockSpec((1, D), lambda b: (0, 0)),
            pl.BlockSpec((D, D), lambda b: (0, 0)),
            pl.BlockSpec((1, D), lambda b: (0, 0)),
            pl.BlockSpec((D, 1), lambda b: (0, 0)),
            pl.BlockSpec(memory_space=pltpu.MemorySpace.SMEM),
        ],
        out_specs=[
            pl.BlockSpec((BB, D), lambda b: (b, 0)),
            pl.BlockSpec(memory_space=pltpu.MemorySpace.HBM),
            pl.BlockSpec((BB, M + 1), lambda b: (b, 0)),
            pl.BlockSpec((BB, 1), lambda b: (b, 0)),
        ],
        out_shape=[
            jax.ShapeDtypeStruct((B, D), jnp.float32),
            jax.ShapeDtypeStruct((B, M + 1, D), jnp.float32),
            jax.ShapeDtypeStruct((B, M + 1), jnp.int32),
            jax.ShapeDtypeStruct((B, 1), jnp.int32),
        ],
        scratch_shapes=[
            pltpu.VMEM((BB, 1, D), jnp.float32),
            pltpu.SemaphoreType.DMA((NCH,)),
            pltpu.SemaphoreType.DMA,
        ],
        compiler_params=pltpu.CompilerParams(
            dimension_semantics=("arbitrary",),
        ),
    )(feat_mem, feat_mem, feat_mem, feat_mem, cls, freq_mem,
      wvT, bv, wuT, bu, wa, ba)

    min_new = pl.pallas_call(
        _min_body,
        grid=(NB,),
        in_specs=[
            pl.BlockSpec((BB, M), lambda b: (b, 0)),
            pl.BlockSpec((BB, 1), lambda b: (b, 0)),
        ],
        out_specs=pl.BlockSpec((BB, M + 1), lambda b: (b, 0)),
        out_shape=jax.ShapeDtypeStruct((B, M + 1), jnp.int32),
    )(min_mem, idx)

    return z, feat, freq_new, min_new


# speedup vs baseline: 1.0724x; 1.0724x over previous
import functools
import jax
import jax.numpy as jnp
from jax import lax
from jax.experimental import pallas as pl
from jax.experimental.pallas import tpu as pltpu

B, M, D = 256, 2048, 128
BB = 8
NB = B // BB
BIG = 2 ** 30


def _attn_body(feat_ref, cls_ref, freq_ref, wv_ref, bv_ref,
               wu_ref, bu_ref, wa_ref, ba_ref,
               z_ref, feat_out_hbm, freq_out_ref, idx_ref,
               cls_stage, sem):
    b = pl.program_id(0)
    rows = pl.ds(b * BB, BB)

    bulk = pltpu.make_async_copy(
        feat_ref, feat_out_hbm.at[rows, pl.ds(0, M)], sem)
    bulk.start()
    c = cls_ref[...]
    cls_stage[...] = c[:, None, :]
    tail = pltpu.make_async_copy(
        cls_stage, feat_out_hbm.at[rows, pl.ds(M, 1)], sem)
    tail.start()

    wv = wv_ref[...]
    wu = wu_ref[...]
    wa = wa_ref[...]
    bv = bv_ref[...]
    bu = bu_ref[...]
    ba = ba_ref[0, 0]

    f = feat_ref[...]
    f2 = f.reshape(BB * M, D)
    v = jnp.tanh(jnp.dot(f2, wv, preferred_element_type=jnp.float32) + bv)
    u = jax.nn.sigmoid(jnp.dot(f2, wu, preferred_element_type=jnp.float32) + bu)
    logit = jnp.dot(v * u, wa, preferred_element_type=jnp.float32) + ba
    score = jax.nn.sigmoid(logit).reshape(BB, M)

    vc = jnp.tanh(jnp.dot(c, wv, preferred_element_type=jnp.float32) + bv)
    uc = jax.nn.sigmoid(jnp.dot(c, wu, preferred_element_type=jnp.float32) + bu)
    sc = jax.nn.sigmoid(jnp.dot(vc * uc, wa,
                                preferred_element_type=jnp.float32) + ba)

    zc = lax.dot_general(score.reshape(BB, 1, M), f,
                         (((2,), (1,)), ((0,), (0,))),
                         preferred_element_type=jnp.float32)
    z_ref[...] = zc.reshape(BB, D) + sc * c

    gmin = jnp.min(score, axis=1, keepdims=True)
    iot = lax.broadcasted_iota(jnp.int32, (BB, M), 1)
    arg = jnp.min(jnp.where(score == gmin, iot, BIG), axis=1, keepdims=True)
    idx_ref[...] = jnp.where(sc < gmin, jnp.int32(M), arg)

    freq_out_ref[:, :M] = freq_ref[...] + 1
    freq_out_ref[:, M:] = jnp.ones((BB, 1), jnp.int32)

    bulk.wait()
    tail.wait()


def _min_body(min_ref, idx_ref, out_ref):
    idx = idx_ref[...]
    iot = lax.broadcasted_iota(jnp.int32, (BB, M), 1)
    out_ref[:, :M] = min_ref[...] + (iot == idx).astype(jnp.int32)
    out_ref[:, M:] = (idx == M).astype(jnp.int32)


def kernel(x, feat_mem, freq_mem, min_mem, is_last, W_v, b_v, W_u, b_u, W_a, b_a):
    cls = x[:, 0, :]
    wvT = W_v.T
    wuT = W_u.T
    wa = W_a.reshape(D, 1)
    ba = b_a.reshape(1, 1)
    bv = b_v.reshape(1, D)
    bu = b_u.reshape(1, D)

    z, feat, freq_new, idx = pl.pallas_call(
        _attn_body,
        grid=(NB,),
        in_specs=[
            pl.BlockSpec((BB, M, D), lambda b: (b, 0, 0)),
            pl.BlockSpec((BB, D), lambda b: (b, 0)),
            pl.BlockSpec((BB, M), lambda b: (b, 0)),
            pl.BlockSpec((D, D), lambda b: (0, 0)),
            pl.BlockSpec((1, D), lambda b: (0, 0)),
            pl.BlockSpec((D, D), lambda b: (0, 0)),
            pl.BlockSpec((1, D), lambda b: (0, 0)),
            pl.BlockSpec((D, 1), lambda b: (0, 0)),
            pl.BlockSpec(memory_space=pltpu.MemorySpace.SMEM),
        ],
        out_specs=[
            pl.BlockSpec((BB, D), lambda b: (b, 0)),
            pl.BlockSpec(memory_space=pltpu.MemorySpace.HBM),
            pl.BlockSpec((BB, M + 1), lambda b: (b, 0)),
            pl.BlockSpec((BB, 1), lambda b: (b, 0)),
        ],
        out_shape=[
            jax.ShapeDtypeStruct((B, D), jnp.float32),
            jax.ShapeDtypeStruct((B, M + 1, D), jnp.float32),
            jax.ShapeDtypeStruct((B, M + 1), jnp.int32),
            jax.ShapeDtypeStruct((B, 1), jnp.int32),
        ],
        scratch_shapes=[
            pltpu.VMEM((BB, 1, D), jnp.float32),
            pltpu.SemaphoreType.DMA,
        ],
        compiler_params=pltpu.CompilerParams(
            dimension_semantics=("arbitrary",),
        ),
    )(feat_mem, cls, freq_mem, wvT, bv, wuT, bu, wa, ba)

    min_new = pl.pallas_call(
        _min_body,
        grid=(NB,),
        in_specs=[
            pl.BlockSpec((BB, M), lambda b: (b, 0)),
            pl.BlockSpec((BB, 1), lambda b: (b, 0)),
        ],
        out_specs=pl.BlockSpec((BB, M + 1), lambda b: (b, 0)),
        out_shape=jax.ShapeDtypeStruct((B, M + 1), jnp.int32),
    )(min_mem, idx)

    return z, feat, freq_new, min_new


# speedup vs baseline: 1.0827x; 1.0096x over previous
import functools
import jax
import jax.numpy as jnp
from jax import lax
from jax.experimental import pallas as pl
from jax.experimental.pallas import tpu as pltpu

B, M, D = 256, 2048, 128
BB = 8
NB = B // BB
BIG = 2 ** 30


def _attn_body(feat_ref, cls_ref, freq_ref, min_ref, wv_ref, bv_ref,
               wu_ref, bu_ref, wa_ref, ba_ref,
               z_ref, feat_out_hbm, freq_out_ref, min_out_ref,
               cls_stage, sem):
    b = pl.program_id(0)
    rows = pl.ds(b * BB, BB)

    bulk = pltpu.make_async_copy(
        feat_ref, feat_out_hbm.at[rows, pl.ds(0, M)], sem)
    bulk.start()
    c = cls_ref[...]
    cls_stage[...] = c[:, None, :]
    tail = pltpu.make_async_copy(
        cls_stage, feat_out_hbm.at[rows, pl.ds(M, 1)], sem)
    tail.start()

    wv = wv_ref[...]
    wu = wu_ref[...]
    wa = wa_ref[...]
    bv = bv_ref[...]
    bu = bu_ref[...]
    ba = ba_ref[0, 0]

    f = feat_ref[...]
    f2 = f.reshape(BB * M, D)
    v = jnp.tanh(jnp.dot(f2, wv, preferred_element_type=jnp.float32) + bv)
    u = jax.nn.sigmoid(jnp.dot(f2, wu, preferred_element_type=jnp.float32) + bu)
    logit = jnp.dot(v * u, wa, preferred_element_type=jnp.float32) + ba
    score = jax.nn.sigmoid(logit).reshape(BB, M)

    vc = jnp.tanh(jnp.dot(c, wv, preferred_element_type=jnp.float32) + bv)
    uc = jax.nn.sigmoid(jnp.dot(c, wu, preferred_element_type=jnp.float32) + bu)
    sc = jax.nn.sigmoid(jnp.dot(vc * uc, wa,
                                preferred_element_type=jnp.float32) + ba)

    zc = lax.dot_general(score.reshape(BB, 1, M), f,
                         (((2,), (1,)), ((0,), (0,))),
                         preferred_element_type=jnp.float32)
    z_ref[...] = zc.reshape(BB, D) + sc * c

    gmin = jnp.min(score, axis=1, keepdims=True)
    iot = lax.broadcasted_iota(jnp.int32, (BB, M), 1)
    arg = jnp.min(jnp.where(score == gmin, iot, BIG), axis=1, keepdims=True)
    idx = jnp.where(sc < gmin, jnp.int32(M), arg)

    freq_out_ref[:, :M] = freq_ref[...] + 1
    freq_out_ref[:, M:] = jnp.ones((BB, 1), jnp.int32)
    min_out_ref[:, :M] = min_ref[...] + (iot == idx).astype(jnp.int32)
    min_out_ref[:, M:] = (idx == M).astype(jnp.int32)

    bulk.wait()
    tail.wait()


def kernel(x, feat_mem, freq_mem, min_mem, is_last, W_v, b_v, W_u, b_u, W_a, b_a):
    cls = x[:, 0, :]
    wvT = W_v.T
    wuT = W_u.T
    wa = W_a.reshape(D, 1)
    ba = b_a.reshape(1, 1)
    bv = b_v.reshape(1, D)
    bu = b_u.reshape(1, D)

    z, feat, freq_new, min_new = pl.pallas_call(
        _attn_body,
        grid=(NB,),
        in_specs=[
            pl.BlockSpec((BB, M, D), lambda b: (b, 0, 0)),
            pl.BlockSpec((BB, D), lambda b: (b, 0)),
            pl.BlockSpec((BB, M), lambda b: (b, 0)),
            pl.BlockSpec((BB, M), lambda b: (b, 0)),
            pl.BlockSpec((D, D), lambda b: (0, 0)),
            pl.BlockSpec((1, D), lambda b: (0, 0)),
            pl.BlockSpec((D, D), lambda b: (0, 0)),
            pl.BlockSpec((1, D), lambda b: (0, 0)),
            pl.BlockSpec((D, 1), lambda b: (0, 0)),
            pl.BlockSpec(memory_space=pltpu.MemorySpace.SMEM),
        ],
        out_specs=[
            pl.BlockSpec((BB, D), lambda b: (b, 0)),
            pl.BlockSpec(memory_space=pltpu.MemorySpace.HBM),
            pl.BlockSpec((BB, M + 1), lambda b: (b, 0)),
            pl.BlockSpec((BB, M + 1), lambda b: (b, 0)),
        ],
        out_shape=[
            jax.ShapeDtypeStruct((B, D), jnp.float32),
            jax.ShapeDtypeStruct((B, M + 1, D), jnp.float32),
            jax.ShapeDtypeStruct((B, M + 1), jnp.int32),
            jax.ShapeDtypeStruct((B, M + 1), jnp.int32),
        ],
        scratch_shapes=[
            pltpu.VMEM((BB, 1, D), jnp.float32),
            pltpu.SemaphoreType.DMA,
        ],
        compiler_params=pltpu.CompilerParams(
            dimension_semantics=("arbitrary",),
        ),
    )(feat_mem, cls, freq_mem, min_mem, wvT, bv, wuT, bu, wa, ba)

    return z, feat, freq_new, min_new
